# baseline (speedup 1.0000x reference)
import functools

import jax
import jax.numpy as jnp
from jax import lax
from jax.experimental import pallas as pl
from jax.experimental.pallas import tpu as pltpu
from jax.experimental.pallas import tpu_sc as plsc

NBUF = 3
NOUT = 2
CHUNK = 128


@functools.lru_cache(maxsize=None)
def _build_sc_embed(BL, L, D):
    info = plsc.get_sparse_core_info()
    NC, NS = info.num_cores, info.num_subcores
    NW = NC * NS
    assert D == 64 and L % 8 == 0
    per_w = BL // NW
    assert BL % (NW * CHUNK) == 0 and per_w % L == 0
    n_chunks = per_w // CHUNK
    period = NBUF * NOUT
    assert (n_chunks - 2) % period == 0 and n_chunks >= period + 2
    n_packs = (n_chunks - 2) // period
    mesh = plsc.VectorSubcoreMesh(core_axis_name="c", subcore_axis_name="s")

    @functools.partial(
        pl.kernel,
        mesh=mesh,
        out_type=jax.ShapeDtypeStruct((BL, D), jnp.float32),
        scratch_types=(
            [pltpu.VMEM((n_chunks, CHUNK), jnp.int32),
             pltpu.VMEM((NBUF, CHUNK), jnp.int32),
             pltpu.VMEM((L // 2, 2 * D), jnp.float32)]
            + [pltpu.VMEM((CHUNK, 2 * D), jnp.float32) for _ in range(NBUF)]
            + [pltpu.VMEM((CHUNK, D), jnp.float32) for _ in range(NOUT)]
            + [pltpu.SemaphoreType.DMA for _ in range(NBUF + NOUT)]
        ),
    )
    def embed(x_hbm, tok_hbm, pos_hbm, out_hbm, idx_v, idx2, pos_v, *refs):
        gath = refs[:NBUF]
        outb = refs[NBUF:NBUF + NOUT]
        gsem = refs[NBUF + NOUT:2 * NBUF + NOUT]
        ssem = refs[2 * NBUF + NOUT:]
        wid = lax.axis_index("s") * NC + lax.axis_index("c")
        base = wid * per_w

        pltpu.sync_copy(x_hbm.at[wid], idx_v)
        pltpu.sync_copy(pos_hbm, pos_v)

        def prep_and_gather(g, b):
            def sh(q, c):
                sl = pl.ds(q * 16, 16)
                idx2[b, sl] = lax.shift_right_logical(idx_v[g, sl], 1)
                return c
            lax.fori_loop(0, CHUNK // 16, sh, 0, unroll=8)
            pltpu.make_async_copy(
                tok_hbm.at[idx2.at[b]], gath[b], gsem[b]).start()

        def wait_gather(b):
            pltpu.make_async_copy(
                tok_hbm.at[idx2.at[b]], gath[b], gsem[b]).wait()

        def start_store(g, o):
            pltpu.make_async_copy(
                outb[o], out_hbm.at[pl.ds(base + g * CHUNK, CHUNK)],
                ssem[o]).start()

        def wait_store(o):
            pltpu.make_async_copy(
                outb[o], out_hbm.at[pl.ds(base, CHUNK)], ssem[o]).wait()

        def compute(g, b, o):
            p0 = lax.rem(g * CHUNK, L)

            def blk(ii, p):
                tvec = idx_v[g, pl.ds(ii * 16, 16)]
                for j in range(16):
                    i = ii * 16 + j
                    tc = lax.shift_left(tvec[j] & 1, 6)
                    ph = lax.shift_right_logical(p, 1)
                    pc = lax.shift_left(p & 1, 6)
                    for q in range(D // 16):
                        outb[o][i, pl.ds(q * 16, 16)] = (
                            gath[b][i, pl.ds(tc + q * 16, 16)]
                            + pos_v[ph, pl.ds(pc + q * 16, 16)])
                    p = jnp.where(p + 1 == L, 0, p + 1)
                return p

            lax.fori_loop(0, CHUNK // 16, blk, p0)

        def body(g, b, o, prefetch, store_wait):
            if prefetch:
                prep_and_gather(g + 2, (b + 2) % NBUF)
            wait_gather(b)
            if store_wait:
                wait_store(o)
            compute(g, b, o)
            start_store(g, o)

        prep_and_gather(0, 0)
        prep_and_gather(1, 1)
        body(0, 0, 0, True, False)
        body(1, 1, 1, True, False)

        def pack(pk, c):
            g0 = pk * period + 2
            for j in range(period):
                body(g0 + j, (2 + j) % NBUF, j % NOUT, True, True)
            return c

        lax.fori_loop(0, n_packs - 1, pack, 0)

        g0 = (n_packs - 1) * period + 2
        for j in range(period):
            body(g0 + j, (2 + j) % NBUF, j % NOUT,
                 g0 + j + 2 < n_chunks, True)
        for o in range(NOUT):
            wait_store(o)

    return embed


def kernel(x, token_table, pos_table):
    B, L = x.shape
    V, D = token_table.shape
    BL = B * L
    info = plsc.get_sparse_core_info()
    NW = info.num_cores * info.num_subcores
    x_r = x.astype(jnp.int32).reshape(NW, BL // (NW * CHUNK), CHUNK)
    tok2 = token_table.reshape(V // 2, 2 * D)
    pos2 = pos_table.reshape(L // 2, 2 * D)
    out = _build_sc_embed(BL, L, D)(x_r, tok2, pos2)
    return out.reshape(B, L, D)
